# Initial kernel scaffold; baseline (speedup 1.0000x reference)
import jax
import jax.numpy as jnp
from jax.experimental import pallas as pl


def kernel(mem, idx, val):
    raise NotImplementedError("write your pallas kernel here")


# speedup vs baseline: 5.4382x; 5.4382x over previous
import functools

import jax
import jax.numpy as jnp
from jax import lax
from jax.experimental import pallas as pl
from jax.experimental.pallas import tpu as pltpu
from jax.experimental.pallas import tpu_sc as plsc

NC = 2
NS = 16
NW = NC * NS
L = 16


def _i32(x):
    return jnp.full((L,), x, jnp.int32)


def _make_sc_scatter(M, D, B, cap, sec, chunk, interpret=False):
    rows_per_w = M // NW
    nsec = B // sec
    ncmax = cap // chunk

    def body(out_hbm, idx_hbm, val_hbm,
             idx_buf, tgt_c, pos_c, kt_flat, kp_flat,
             winner, kt2d, kp2d, vbuf, sem_g, sem_s):
        wid = lax.axis_index("s") * NC + lax.axis_index("c")
        lo = wid * rows_per_w
        lo_v = _i32(lo)
        hi_v = lo_v + _i32(rows_per_w)
        iota = lax.iota(jnp.int32, L)

        neg1 = _i32(-1)

        def init_body(i, _):
            winner[pl.ds(i * L, L)] = neg1
            return 0

        lax.fori_loop(0, rows_per_w // L + 1, init_body, 0)

        def sec_body(s, cnt):
            pltpu.sync_copy(idx_hbm.at[pl.ds(s * sec, sec)], idx_buf)

            def scan_body(i, c):
                v = idx_buf[pl.ds(i * L, L)]
                bvec = _i32(s * sec) + _i32(i * L) + iota
                m = (v >= lo_v) & (v < hi_v)
                m = m & (_i32(c) < _i32(cap - L))
                mi = m.astype(jnp.int32)
                dest = _i32(c) + plsc.cumsum(mi) - mi
                plsc.store_scatter(tgt_c, [dest], v, mask=m)
                plsc.store_scatter(pos_c, [dest], bvec, mask=m)
                return c + jnp.sum(mi)

            return lax.fori_loop(0, sec // L, scan_body, cnt)

        cnt = lax.fori_loop(0, nsec, sec_body, jnp.int32(0))

        nv = (cnt + L - 1) // L

        def score_body(i, _):
            base = i * L
            t = tgt_c[pl.ds(base, L)]
            lanepos = _i32(base) + iota
            valid = lanepos < _i32(cnt)
            tl = jnp.clip(t - lo_v, 0, rows_per_w - 1)
            tmod = jnp.where(valid, tl, -1 - iota)
            dup = tmod != tmod
            for s in range(1, L):
                perm = jnp.minimum(iota + s, L - 1)
                dup = dup | ((tmod[perm] == tmod) & (iota + _i32(s) < _i32(L)))
            wmask = valid & (~dup)
            plsc.store_scatter(winner, [tl], lanepos, mask=wmask)
            return 0

        lax.fori_loop(0, nv, score_body, 0)

        def keep_body(i, c2):
            base = i * L
            t = tgt_c[pl.ds(base, L)]
            p = pos_c[pl.ds(base, L)]
            lanepos = _i32(base) + iota
            valid = lanepos < _i32(cnt)
            tl = jnp.clip(t - lo_v, 0, rows_per_w - 1)
            w = plsc.load_gather(winner, [tl], mask=valid)
            keep = valid & (w == lanepos)
            ki = keep.astype(jnp.int32)
            dest = _i32(c2) + plsc.cumsum(ki) - ki
            plsc.store_scatter(kt_flat, [dest], t, mask=keep)
            plsc.store_scatter(kp_flat, [dest], p, mask=keep)
            return c2 + jnp.sum(ki)

        cnt2 = lax.fori_loop(0, nv, keep_body, jnp.int32(0))

        jm = jnp.maximum(cnt2 - 1, 0)
        jb = (jm // L) * L
        sel_l = iota == _i32(jm - jb)
        vt = kt_flat[pl.ds(jb, L)]
        vp = kp_flat[pl.ds(jb, L)]
        last_t = _i32(jnp.sum(jnp.where(sel_l, vt, 0)))
        last_p = _i32(jnp.sum(jnp.where(sel_l, vp, 0)))
        start = (cnt2 // L) * L
        for k in range(chunk // L):
            off = start + k * L
            posv = _i32(off) + iota
            sel = posv >= _i32(cnt2)
            kt_flat[pl.ds(off, L)] = jnp.where(sel, last_t,
                                               kt_flat[pl.ds(off, L)])
            kp_flat[pl.ds(off, L)] = jnp.where(sel, last_p,
                                               kp_flat[pl.ds(off, L)])

        nc2 = (cnt2 + chunk - 1) // chunk

        def cp_body(c, _):
            for k in range(chunk // L):
                kt2d[c, pl.ds(k * L, L)] = kt_flat[pl.ds(c * chunk + k * L, L)]
                kp2d[c, pl.ds(k * L, L)] = kp_flat[pl.ds(c * chunk + k * L, L)]
            return 0

        lax.fori_loop(0, nc2, cp_body, 0)

        def dma_body(c, _):
            pltpu.async_copy(val_hbm.at[kp2d.at[c]], vbuf, sem_g).wait()
            pltpu.async_copy(vbuf, out_hbm.at[kt2d.at[c]], sem_s).wait()
            return 0

        lax.fori_loop(0, nc2, dma_body, 0)

    mesh = plsc.VectorSubcoreMesh(core_axis_name="c", subcore_axis_name="s")
    return pl.kernel(
        body,
        out_type=(),
        mesh=mesh,
        interpret=interpret,
        compiler_params=pltpu.CompilerParams(
            needs_layout_passes=False,
            use_tc_tiling_on_sc=False,
        ),
        scratch_types=[
            pltpu.VMEM((sec,), jnp.int32),
            pltpu.VMEM((cap + L,), jnp.int32),
            pltpu.VMEM((cap + L,), jnp.int32),
            pltpu.VMEM((cap + chunk,), jnp.int32),
            pltpu.VMEM((cap + chunk,), jnp.int32),
            pltpu.VMEM((rows_per_w + L,), jnp.int32),
            pltpu.VMEM((ncmax, chunk), jnp.int32),
            pltpu.VMEM((ncmax, chunk), jnp.int32),
            pltpu.VMEM((chunk, D), jnp.float32),
            pltpu.SemaphoreType.DMA,
            pltpu.SemaphoreType.DMA,
        ],
    )


def _scatter_impl(mem, idx, val, interpret=False):
    M, D = mem.shape
    B = idx.shape[0]
    sc_scatter = _make_sc_scatter(M, D, B, cap=6144, sec=16384, chunk=128,
                                  interpret=interpret)
    out_ref = jax.new_ref(mem)
    sc_scatter(out_ref, idx.astype(jnp.int32), val)
    return out_ref[...]


@jax.jit
def kernel(mem, idx, val):
    return _scatter_impl(mem, idx, val)


# speedup vs baseline: 8.0012x; 1.4713x over previous
import jax
import jax.numpy as jnp
from jax import lax
from jax.experimental import pallas as pl
from jax.experimental.pallas import tpu as pltpu
from jax.experimental.pallas import tpu_sc as plsc

NC = 2
NS = 16
NW = NC * NS
L = 16

MC = 1000000
DD = 64
BB = 65536
TILE = 128
NT_FULL = MC // TILE
PART_BASE = NT_FULL * TILE
PART_W = MC - PART_BASE
PART_OWNER = NT_FULL % NW
PART_BIN = NT_FULL // NW
SEC = 16384
NSEC = BB // SEC
CAP = 6144
NBINS = 256


def _i32(x):
    return jnp.full((L,), x, jnp.int32)


def _sc_body(memT, idx_hbm, val_hbm, outT,
             idx_buf, tgt_c, pos_c, binned_t, binned_p,
             runhist, binbase, nextfree, chunk, chunk64, vbuf, st16,
             sem_i, sem_o, sem_g):
    wid = lax.axis_index("s") * NC + lax.axis_index("c")
    iota = lax.iota(jnp.int32, L)
    widv = _i32(wid)

    def sec_body(s, cnt):
        pltpu.sync_copy(idx_hbm.at[pl.ds(s * SEC, SEC)], idx_buf)

        def scan_body(i, c):
            v = idx_buf[pl.ds(i * L, L)]
            bvec = _i32(s * SEC) + _i32(i * L) + iota
            m = (lax.shift_right_logical(v, _i32(7)) & _i32(NW - 1)) == widv
            m = m & (_i32(c) < _i32(CAP - L))
            mi = jnp.where(m, _i32(1), _i32(0))
            dest = _i32(c) + plsc.cumsum(mi) - mi
            plsc.store_scatter(tgt_c, [dest], v, mask=m)
            plsc.store_scatter(pos_c, [dest], bvec, mask=m)
            return c + jnp.sum(mi)

        return lax.fori_loop(0, SEC // L, scan_body, cnt)

    cnt = lax.fori_loop(0, NSEC, sec_body, jnp.int32(0))
    nv = (cnt + L - 1) // L

    def zb(j, _):
        runhist[pl.ds(j * L, L)] = _i32(0)
        return 0

    lax.fori_loop(0, NBINS // L, zb, 0)

    lane0 = iota == _i32(0)

    def hist_body(i, _):
        tb = plsc.load_gather(tgt_c, [_i32(i)])
        binv = jnp.clip(lax.shift_right_logical(tb, _i32(12)), 0, NBINS - 1)
        plsc.addupdate_scatter(runhist, [binv], _i32(1), mask=lane0)
        return 0

    lax.fori_loop(0, cnt, hist_body, 0)

    def scan_bins(j, run):
        h = runhist[pl.ds(j * L, L)]
        c = plsc.cumsum(h)
        binbase[pl.ds(j * L, L)] = _i32(run) + c - h
        return run + jnp.sum(h)

    lax.fori_loop(0, NBINS // L, scan_bins, jnp.int32(0))

    def cpnf(j, _):
        nextfree[pl.ds(j * L, L)] = binbase[pl.ds(j * L, L)]
        return 0

    lax.fori_loop(0, NBINS // L, cpnf, 0)

    def place_body(i, _):
        tb = plsc.load_gather(tgt_c, [_i32(i)])
        pb = plsc.load_gather(pos_c, [_i32(i)])
        binv = jnp.clip(lax.shift_right_logical(tb, _i32(12)), 0, NBINS - 1)
        dest = plsc.load_gather(nextfree, [binv])
        plsc.store_scatter(binned_t, [dest], tb, mask=lane0)
        plsc.store_scatter(binned_p, [dest], pb, mask=lane0)
        plsc.addupdate_scatter(nextfree, [binv], _i32(1), mask=lane0)
        return 0

    lax.fori_loop(0, cnt, place_body, 0)

    def bin_bounds(k):
        hb = (k // L) * L
        hv = runhist[pl.ds(hb, L)]
        bv = binbase[pl.ds(hb, L)]
        selk = iota == _i32(k - hb)
        n_k = jnp.sum(jnp.where(selk, hv, _i32(0)))
        b_k = jnp.sum(jnp.where(selk, bv, _i32(0)))
        return n_k, b_k

    def apply_updates(k, cbuf):
        n_k, b_k = bin_bounds(k)
        nrounds = (n_k + L - 1) // L

        def round_body(r, _2):
            off = b_k + r * L
            li = _i32(off) + jnp.minimum(iota, _i32(n_k - r * L - 1))
            col16 = plsc.load_gather(binned_t, [li])
            pos16 = plsc.load_gather(binned_p, [li])
            st16[pl.ds(0, L)] = pos16
            pltpu.async_copy(val_hbm.at[st16], vbuf, sem_g).wait()
            cl16 = col16 & _i32(TILE - 1)
            m_sc = jnp.minimum(n_k - r * L, L)

            def ent_body(i, _3):
                clv = jnp.sum(jnp.where(iota == _i32(i), cl16, _i32(0)))
                for kq in range(DD // L):
                    x = vbuf[i, pl.ds(kq * L, L)]
                    plsc.store_scatter(
                        cbuf, [iota + _i32(kq * L), _i32(clv)], x)
                return 0

            lax.fori_loop(0, m_sc, ent_body, 0)
            return 0

        lax.fori_loop(0, nrounds, round_body, 0)

    my_nt = (NT_FULL - wid + NW - 1) // NW

    def tile_body(k, _):
        t = wid + k * NW
        base = t * TILE
        pltpu.async_copy(memT.at[:, pl.ds(base, TILE)], chunk, sem_i).wait()
        apply_updates(k, chunk)
        pltpu.async_copy(chunk, outT.at[:, pl.ds(base, TILE)], sem_o).wait()
        return 0

    lax.fori_loop(0, my_nt, tile_body, 0)

    def part_body(_, _2):
        pltpu.async_copy(memT.at[:, pl.ds(PART_BASE, PART_W)], chunk64,
                         sem_i).wait()
        apply_updates(PART_BIN, chunk64)
        pltpu.async_copy(chunk64, outT.at[:, pl.ds(PART_BASE, PART_W)],
                         sem_o).wait()
        return 0

    is_owner = jnp.where(wid == PART_OWNER, 1, 0)
    lax.fori_loop(0, is_owner, part_body, 0)


def _make_sc_update():
    mesh = plsc.VectorSubcoreMesh(core_axis_name="c", subcore_axis_name="s")
    return pl.kernel(
        _sc_body,
        out_type=jax.ShapeDtypeStruct((DD, MC), jnp.float32),
        mesh=mesh,
        compiler_params=pltpu.CompilerParams(needs_layout_passes=False),
        scratch_types=[
            pltpu.VMEM((SEC,), jnp.int32),
            pltpu.VMEM((CAP,), jnp.int32),
            pltpu.VMEM((CAP,), jnp.int32),
            pltpu.VMEM((CAP,), jnp.int32),
            pltpu.VMEM((CAP,), jnp.int32),
            pltpu.VMEM((NBINS,), jnp.int32),
            pltpu.VMEM((NBINS,), jnp.int32),
            pltpu.VMEM((NBINS,), jnp.int32),
            pltpu.VMEM((DD, TILE), jnp.float32),
            pltpu.VMEM((DD, PART_W), jnp.float32),
            pltpu.VMEM((L, TILE), jnp.float32),
            pltpu.VMEM((L,), jnp.int32),
            pltpu.SemaphoreType.DMA,
            pltpu.SemaphoreType.DMA,
            pltpu.SemaphoreType.DMA,
        ],
    )


@jax.jit
def kernel(mem, idx, val):
    idx32 = idx.astype(jnp.int32)
    memT = mem.T
    val128 = jnp.pad(val, ((0, 0), (0, TILE - DD)))
    outT = _make_sc_update()(memT, idx32, val128)
    return outT.T


# speedup vs baseline: 11.3958x; 1.4243x over previous
import jax
import jax.numpy as jnp
from jax import lax
from jax.experimental import pallas as pl
from jax.experimental.pallas import tpu as pltpu
from jax.experimental.pallas import tpu_sc as plsc

NC = 2
NS = 16
NW = NC * NS
L = 16

MC = 1000000
DD = 64
BB = 65536
GROUP = 512
NT_FULL = MC // GROUP
PART_BASE = NT_FULL * GROUP
PART_W = MC - PART_BASE
PART_OWNER = NT_FULL % NW
PART_BIN = NT_FULL // NW
OW_SHIFT = 9
BIN_SHIFT = 14
SEC = 16384
NSEC = BB // SEC
CAP = 6144
NBINS = 64


def _i32(x):
    return jnp.full((L,), x, jnp.int32)


def _sc_body(memT, idx_hbm, val_hbm, outT,
             idx_buf, tgt_c, pos_c, binned_t, binned_p,
             runhist, binbase, nextfree, chunk, chunk64, vbuf, st16,
             sem_i, sem_o, sem_g):
    wid = lax.axis_index("s") * NC + lax.axis_index("c")
    iota = lax.iota(jnp.int32, L)
    widv = _i32(wid)

    def sec_body(s, cnt):
        pltpu.sync_copy(idx_hbm.at[pl.ds(s * SEC, SEC)], idx_buf)

        def scan_body(i, c):
            v = idx_buf[pl.ds(i * L, L)]
            bvec = _i32(s * SEC) + _i32(i * L) + iota
            m = (lax.shift_right_logical(v, _i32(OW_SHIFT)) & _i32(NW - 1)) == widv
            m = m & (_i32(c) < _i32(CAP - L))
            mi = jnp.where(m, _i32(1), _i32(0))
            dest = _i32(c) + plsc.cumsum(mi) - mi
            plsc.store_scatter(tgt_c, [dest], v, mask=m)
            plsc.store_scatter(pos_c, [dest], bvec, mask=m)
            return c + jnp.sum(mi)

        return lax.fori_loop(0, SEC // L, scan_body, cnt)

    cnt = lax.fori_loop(0, NSEC, sec_body, jnp.int32(0))
    nv = (cnt + L - 1) // L

    def zb(j, _):
        runhist[pl.ds(j * L, L)] = _i32(0)
        return 0

    lax.fori_loop(0, NBINS // L, zb, 0)

    lane0 = iota == _i32(0)

    def hist_body(i, _):
        tb = plsc.load_gather(tgt_c, [_i32(i)])
        binv = jnp.clip(lax.shift_right_logical(tb, _i32(BIN_SHIFT)), 0, NBINS - 1)
        plsc.addupdate_scatter(runhist, [binv], _i32(1), mask=lane0)
        return 0

    lax.fori_loop(0, cnt, hist_body, 0)

    def scan_bins(j, run):
        h = runhist[pl.ds(j * L, L)]
        c = plsc.cumsum(h)
        binbase[pl.ds(j * L, L)] = _i32(run) + c - h
        return run + jnp.sum(h)

    lax.fori_loop(0, NBINS // L, scan_bins, jnp.int32(0))

    def cpnf(j, _):
        nextfree[pl.ds(j * L, L)] = binbase[pl.ds(j * L, L)]
        return 0

    lax.fori_loop(0, NBINS // L, cpnf, 0)

    def place_body(i, _):
        tb = plsc.load_gather(tgt_c, [_i32(i)])
        pb = plsc.load_gather(pos_c, [_i32(i)])
        binv = jnp.clip(lax.shift_right_logical(tb, _i32(BIN_SHIFT)), 0, NBINS - 1)
        dest = plsc.load_gather(nextfree, [binv])
        plsc.store_scatter(binned_t, [dest], tb, mask=lane0)
        plsc.store_scatter(binned_p, [dest], pb, mask=lane0)
        plsc.addupdate_scatter(nextfree, [binv], _i32(1), mask=lane0)
        return 0

    lax.fori_loop(0, cnt, place_body, 0)

    def bin_bounds(k):
        hb = (k // L) * L
        hv = runhist[pl.ds(hb, L)]
        bv = binbase[pl.ds(hb, L)]
        selk = iota == _i32(k - hb)
        n_k = jnp.sum(jnp.where(selk, hv, _i32(0)))
        b_k = jnp.sum(jnp.where(selk, bv, _i32(0)))
        return n_k, b_k

    def apply_updates(k, cbuf):
        n_k, b_k = bin_bounds(k)
        nrounds = (n_k + L - 1) // L

        def round_body(r, _2):
            off = b_k + r * L
            li = _i32(off) + jnp.minimum(iota, _i32(n_k - r * L - 1))
            col16 = plsc.load_gather(binned_t, [li])
            pos16 = plsc.load_gather(binned_p, [li])
            st16[pl.ds(0, L)] = pos16
            pltpu.async_copy(val_hbm.at[st16], vbuf, sem_g).wait()
            cl16 = col16 & _i32(GROUP - 1)
            m_sc = jnp.minimum(n_k - r * L, L)

            def ent_body(i, _3):
                clv = jnp.sum(jnp.where(iota == _i32(i), cl16, _i32(0)))
                for kq in range(DD // L):
                    x = vbuf[i, pl.ds(kq * L, L)]
                    plsc.store_scatter(
                        cbuf, [iota + _i32(kq * L), _i32(clv)], x)
                return 0

            lax.fori_loop(0, m_sc, ent_body, 0)
            return 0

        lax.fori_loop(0, nrounds, round_body, 0)

    my_nt = (NT_FULL - wid + NW - 1) // NW

    def tile_body(k, _):
        t = wid + k * NW
        base = t * GROUP
        pltpu.async_copy(memT.at[:, pl.ds(base, GROUP)], chunk, sem_i).wait()
        apply_updates(k, chunk)
        pltpu.async_copy(chunk, outT.at[:, pl.ds(base, GROUP)], sem_o).wait()
        return 0

    lax.fori_loop(0, my_nt, tile_body, 0)

    def part_body(_, _2):
        pltpu.async_copy(memT.at[:, pl.ds(PART_BASE, PART_W)], chunk64,
                         sem_i).wait()
        apply_updates(PART_BIN, chunk64)
        pltpu.async_copy(chunk64, outT.at[:, pl.ds(PART_BASE, PART_W)],
                         sem_o).wait()
        return 0

    is_owner = jnp.where(wid == PART_OWNER, 1, 0)
    lax.fori_loop(0, is_owner, part_body, 0)


def _make_sc_update():
    mesh = plsc.VectorSubcoreMesh(core_axis_name="c", subcore_axis_name="s")
    return pl.kernel(
        _sc_body,
        out_type=jax.ShapeDtypeStruct((DD, MC), jnp.float32),
        mesh=mesh,
        compiler_params=pltpu.CompilerParams(needs_layout_passes=False),
        scratch_types=[
            pltpu.VMEM((SEC,), jnp.int32),
            pltpu.VMEM((CAP,), jnp.int32),
            pltpu.VMEM((CAP,), jnp.int32),
            pltpu.VMEM((CAP,), jnp.int32),
            pltpu.VMEM((CAP,), jnp.int32),
            pltpu.VMEM((NBINS,), jnp.int32),
            pltpu.VMEM((NBINS,), jnp.int32),
            pltpu.VMEM((NBINS,), jnp.int32),
            pltpu.VMEM((DD, GROUP), jnp.float32),
            pltpu.VMEM((DD, PART_W), jnp.float32),
            pltpu.VMEM((L, 128), jnp.float32),
            pltpu.VMEM((L,), jnp.int32),
            pltpu.SemaphoreType.DMA,
            pltpu.SemaphoreType.DMA,
            pltpu.SemaphoreType.DMA,
        ],
    )


@jax.jit
def kernel(mem, idx, val):
    idx32 = idx.astype(jnp.int32)
    memT = mem.T
    val128 = jnp.pad(val, ((0, 0), (0, 128 - DD)))
    outT = _make_sc_update()(memT, idx32, val128)
    return outT.T


# speedup vs baseline: 13.6902x; 1.2013x over previous
import jax
import jax.numpy as jnp
from jax import lax
from jax.experimental import pallas as pl
from jax.experimental.pallas import tpu as pltpu
from jax.experimental.pallas import tpu_sc as plsc

NC = 2
NS = 16
NW = NC * NS
L = 16

MC = 1000000
DD = 64
BB = 65536
GROUP = 512
NT_FULL = MC // GROUP
PART_BASE = NT_FULL * GROUP
PART_W = MC - PART_BASE
PART_OWNER = NT_FULL % NW
PART_BIN = NT_FULL // NW
OW_SHIFT = 9
BIN_SHIFT = 14
SEC = 16384
NSEC = BB // SEC
CAP = 6144
NBINS = 64


def _i32(x):
    return jnp.full((L,), x, jnp.int32)


def _sc_body(memT, idx_hbm, val_hbm, outT,
             idx_buf, tgt_c, pos_c, binned_t, binned_p,
             runhist, binbase, nextfree, chunk, chunk2, chunk64, vbuf,
             st16, sem_i, sem_o, sem_i2, sem_o2, sem_g):
    wid = lax.axis_index("s") * NC + lax.axis_index("c")
    iota = lax.iota(jnp.int32, L)
    widv = _i32(wid)

    def sec_body(s, cnt):
        pltpu.sync_copy(idx_hbm.at[pl.ds(s * SEC, SEC)], idx_buf)

        def scan_body(i, c):
            v = idx_buf[pl.ds(i * L, L)]
            bvec = _i32(s * SEC) + _i32(i * L) + iota
            m = (lax.shift_right_logical(v, _i32(OW_SHIFT)) & _i32(NW - 1)) == widv
            m = m & (_i32(c) < _i32(CAP - L))
            mi = jnp.where(m, _i32(1), _i32(0))
            dest = _i32(c) + plsc.cumsum(mi) - mi
            plsc.store_scatter(tgt_c, [dest], v, mask=m)
            plsc.store_scatter(pos_c, [dest], bvec, mask=m)
            return c + jnp.sum(mi)

        return lax.fori_loop(0, SEC // L, scan_body, cnt)

    cnt = lax.fori_loop(0, NSEC, sec_body, jnp.int32(0))
    nv = (cnt + L - 1) // L

    def zb(j, _):
        runhist[pl.ds(j * L, L)] = _i32(0)
        return 0

    lax.fori_loop(0, NBINS // L, zb, 0)

    lane0 = iota == _i32(0)

    def hist_body(i, _):
        tb = plsc.load_gather(tgt_c, [_i32(i)])
        binv = jnp.clip(lax.shift_right_logical(tb, _i32(BIN_SHIFT)), 0, NBINS - 1)
        plsc.addupdate_scatter(runhist, [binv], _i32(1), mask=lane0)
        return 0

    lax.fori_loop(0, cnt, hist_body, 0)

    def scan_bins(j, run):
        h = runhist[pl.ds(j * L, L)]
        c = plsc.cumsum(h)
        binbase[pl.ds(j * L, L)] = _i32(run) + c - h
        return run + jnp.sum(h)

    lax.fori_loop(0, NBINS // L, scan_bins, jnp.int32(0))

    def cpnf(j, _):
        nextfree[pl.ds(j * L, L)] = binbase[pl.ds(j * L, L)]
        return 0

    lax.fori_loop(0, NBINS // L, cpnf, 0)

    def place_body(i, _):
        tb = plsc.load_gather(tgt_c, [_i32(i)])
        pb = plsc.load_gather(pos_c, [_i32(i)])
        binv = jnp.clip(lax.shift_right_logical(tb, _i32(BIN_SHIFT)), 0, NBINS - 1)
        dest = plsc.load_gather(nextfree, [binv])
        plsc.store_scatter(binned_t, [dest], tb, mask=lane0)
        plsc.store_scatter(binned_p, [dest], pb, mask=lane0)
        plsc.addupdate_scatter(nextfree, [binv], _i32(1), mask=lane0)
        return 0

    lax.fori_loop(0, cnt, place_body, 0)

    def bin_bounds(k):
        hb = (k // L) * L
        hv = runhist[pl.ds(hb, L)]
        bv = binbase[pl.ds(hb, L)]
        selk = iota == _i32(k - hb)
        n_k = jnp.sum(jnp.where(selk, hv, _i32(0)))
        b_k = jnp.sum(jnp.where(selk, bv, _i32(0)))
        return n_k, b_k

    def apply_updates(k, cbuf):
        n_k, b_k = bin_bounds(k)
        nrounds = (n_k + L - 1) // L

        def round_body(r, _2):
            off = b_k + r * L
            li = _i32(off) + jnp.minimum(iota, _i32(n_k - r * L - 1))
            col16 = plsc.load_gather(binned_t, [li])
            pos16 = plsc.load_gather(binned_p, [li])
            st16[pl.ds(0, L)] = pos16
            pltpu.async_copy(val_hbm.at[st16], vbuf, sem_g).wait()
            cl16 = col16 & _i32(GROUP - 1)
            m_sc = jnp.minimum(n_k - r * L, L)

            def ent_body(i, _3):
                clv = jnp.sum(jnp.where(iota == _i32(i), cl16, _i32(0)))
                for kq in range(DD // L):
                    x = vbuf[i, pl.ds(kq * L, L)]
                    plsc.store_scatter(
                        cbuf, [iota + _i32(kq * L), _i32(clv)], x)
                return 0

            lax.fori_loop(0, m_sc, ent_body, 0)
            return 0

        lax.fori_loop(0, nrounds, round_body, 0)

    my_nt = (NT_FULL - wid + NW - 1) // NW

    def start_in(k, cbuf, sem):
        t = wid + k * NW
        pltpu.async_copy(memT.at[:, pl.ds(t * GROUP, GROUP)], cbuf, sem)

    def wait_in(cbuf, sem):
        pltpu.make_async_copy(memT.at[:, pl.ds(0, GROUP)], cbuf, sem).wait()

    def start_out(k, cbuf, sem):
        t = wid + k * NW
        pltpu.async_copy(cbuf, outT.at[:, pl.ds(t * GROUP, GROUP)], sem)

    def wait_out(cbuf, sem):
        pltpu.make_async_copy(cbuf, outT.at[:, pl.ds(0, GROUP)], sem).wait()

    def when(cond, fn):
        def b(_, __):
            fn()
            return 0

        lax.fori_loop(0, jnp.where(cond, 1, 0), b, 0)

    start_in(0, chunk, sem_i)
    npairs = (my_nt + 1) // 2

    def pair_body(j, _):
        k0 = j * 2
        k1 = k0 + 1
        when((j > 0) & (k1 < my_nt), lambda: wait_out(chunk2, sem_o2))
        when(k1 < my_nt, lambda: start_in(k1, chunk2, sem_i2))
        wait_in(chunk, sem_i)
        apply_updates(k0, chunk)
        start_out(k0, chunk, sem_o)

        def do_b():
            wait_in(chunk2, sem_i2)
            apply_updates(k1, chunk2)
            start_out(k1, chunk2, sem_o2)

        when(k1 < my_nt, do_b)

        def prefetch_a():
            wait_out(chunk, sem_o)
            start_in(k0 + 2, chunk, sem_i)

        when(k0 + 2 < my_nt, prefetch_a)
        return 0

    lax.fori_loop(0, npairs, pair_body, 0)
    wait_out(chunk, sem_o)
    when(my_nt >= 2, lambda: wait_out(chunk2, sem_o2))

    def part_body(_, _2):
        pltpu.async_copy(memT.at[:, pl.ds(PART_BASE, PART_W)], chunk64,
                         sem_i).wait()
        apply_updates(PART_BIN, chunk64)
        pltpu.async_copy(chunk64, outT.at[:, pl.ds(PART_BASE, PART_W)],
                         sem_o).wait()
        return 0

    is_owner = jnp.where(wid == PART_OWNER, 1, 0)
    lax.fori_loop(0, is_owner, part_body, 0)


def _make_sc_update():
    mesh = plsc.VectorSubcoreMesh(core_axis_name="c", subcore_axis_name="s")
    return pl.kernel(
        _sc_body,
        out_type=jax.ShapeDtypeStruct((DD, MC), jnp.float32),
        mesh=mesh,
        compiler_params=pltpu.CompilerParams(needs_layout_passes=False),
        scratch_types=[
            pltpu.VMEM((SEC,), jnp.int32),
            pltpu.VMEM((CAP,), jnp.int32),
            pltpu.VMEM((CAP,), jnp.int32),
            pltpu.VMEM((CAP,), jnp.int32),
            pltpu.VMEM((CAP,), jnp.int32),
            pltpu.VMEM((NBINS,), jnp.int32),
            pltpu.VMEM((NBINS,), jnp.int32),
            pltpu.VMEM((NBINS,), jnp.int32),
            pltpu.VMEM((DD, GROUP), jnp.float32),
            pltpu.VMEM((DD, GROUP), jnp.float32),
            pltpu.VMEM((DD, PART_W), jnp.float32),
            pltpu.VMEM((L, 128), jnp.float32),
            pltpu.VMEM((L,), jnp.int32),
            pltpu.SemaphoreType.DMA,
            pltpu.SemaphoreType.DMA,
            pltpu.SemaphoreType.DMA,
            pltpu.SemaphoreType.DMA,
            pltpu.SemaphoreType.DMA,
        ],
    )


@jax.jit
def kernel(mem, idx, val):
    idx32 = idx.astype(jnp.int32)
    memT = mem.T
    val128 = jnp.pad(val, ((0, 0), (0, 128 - DD)))
    outT = _make_sc_update()(memT, idx32, val128)
    return outT.T


# speedup vs baseline: 14.5924x; 1.0659x over previous
import jax
import jax.numpy as jnp
from jax import lax
from jax.experimental import pallas as pl
from jax.experimental.pallas import tpu as pltpu
from jax.experimental.pallas import tpu_sc as plsc

NC = 2
NS = 16
NW = NC * NS
L = 16

MC = 1000000
DD = 64
BB = 65536
GROUP = 512
NT_FULL = MC // GROUP
PART_BASE = NT_FULL * GROUP
PART_W = MC - PART_BASE
PART_OWNER = NT_FULL % NW
PART_BIN = NT_FULL // NW
OW_SHIFT = 9
BIN_SHIFT = 14
SEC = 16384
NSEC = BB // SEC
CAP = 6144
NBINS = 64


def _i32(x):
    return jnp.full((L,), x, jnp.int32)


def _sc_body(memT, idx_hbm, val_hbm, outT,
             idx_buf, tgt_c, pos_c, binned_t, binned_p,
             runhist, binbase, nextfree, chunk, chunk2, chunk64, vbuf,
             st16, clbuf, sem_i, sem_o, sem_i2, sem_o2, sem_g):
    wid = lax.axis_index("s") * NC + lax.axis_index("c")
    iota = lax.iota(jnp.int32, L)
    widv = _i32(wid)

    def sec_body(s, cnt):
        pltpu.sync_copy(idx_hbm.at[pl.ds(s * SEC, SEC)], idx_buf)

        def scan_body(i, c):
            for q in range(4):
                off = i * 4 * L + q * L
                v = idx_buf[pl.ds(off, L)]
                bvec = _i32(s * SEC) + _i32(off) + iota
                m = (lax.shift_right_logical(v, _i32(OW_SHIFT))
                     & _i32(NW - 1)) == widv
                m = m & (_i32(c) < _i32(CAP - L))
                mi = jnp.where(m, _i32(1), _i32(0))
                dest = _i32(c) + plsc.cumsum(mi) - mi
                plsc.store_scatter(tgt_c, [dest], v, mask=m)
                plsc.store_scatter(pos_c, [dest], bvec, mask=m)
                c = c + jnp.sum(mi)
            return c

        return lax.fori_loop(0, SEC // (4 * L), scan_body, cnt)

    cnt = lax.fori_loop(0, NSEC, sec_body, jnp.int32(0))
    nv = (cnt + L - 1) // L

    def zb(j, _):
        runhist[pl.ds(j * L, L)] = _i32(0)
        return 0

    lax.fori_loop(0, NBINS // L, zb, 0)

    def hist_body(i, _):
        base = i * L
        t = tgt_c[pl.ds(base, L)]
        valid = (_i32(base) + iota) < _i32(cnt)
        binv = jnp.clip(lax.shift_right_logical(t, _i32(BIN_SHIFT)), 0,
                        NBINS - 1)
        rc, lastm = plsc.scan_count(binv, valid)
        plsc.addupdate_scatter(runhist, [binv], rc, mask=lastm & valid)
        return 0

    lax.fori_loop(0, nv, hist_body, 0)

    def scan_bins(j, run):
        h = runhist[pl.ds(j * L, L)]
        c = plsc.cumsum(h)
        binbase[pl.ds(j * L, L)] = _i32(run) + c - h
        return run + jnp.sum(h)

    lax.fori_loop(0, NBINS // L, scan_bins, jnp.int32(0))

    def cpnf(j, _):
        nextfree[pl.ds(j * L, L)] = _i32(0)
        return 0

    lax.fori_loop(0, NBINS // L, cpnf, 0)

    def place_body(i, _):
        base = i * L
        t = tgt_c[pl.ds(base, L)]
        p = pos_c[pl.ds(base, L)]
        valid = (_i32(base) + iota) < _i32(cnt)
        binv = jnp.clip(lax.shift_right_logical(t, _i32(BIN_SHIFT)), 0,
                        NBINS - 1)
        rc, lastm = plsc.scan_count(binv, valid)
        run = plsc.load_gather(nextfree, [binv], mask=valid)
        bb = plsc.load_gather(binbase, [binv], mask=valid)
        dest = bb + run + rc - _i32(1)
        plsc.store_scatter(binned_t, [dest], t, mask=valid)
        plsc.store_scatter(binned_p, [dest], p, mask=valid)
        plsc.addupdate_scatter(nextfree, [binv], rc, mask=lastm & valid)
        return 0

    lax.fori_loop(0, nv, place_body, 0)

    def bin_bounds(k):
        hb = (k // L) * L
        hv = runhist[pl.ds(hb, L)]
        bv = binbase[pl.ds(hb, L)]
        selk = iota == _i32(k - hb)
        n_k = jnp.sum(jnp.where(selk, hv, _i32(0)))
        b_k = jnp.sum(jnp.where(selk, bv, _i32(0)))
        return n_k, b_k

    def apply_updates(k, cbuf):
        n_k, b_k = bin_bounds(k)
        nrounds = (n_k + L - 1) // L

        def round_body(r, _2):
            off = b_k + r * L
            li = _i32(off) + jnp.minimum(iota, _i32(n_k - r * L - 1))
            col16 = plsc.load_gather(binned_t, [li])
            pos16 = plsc.load_gather(binned_p, [li])
            st16[pl.ds(0, L)] = pos16
            pltpu.async_copy(val_hbm.at[st16], vbuf, sem_g).wait()
            clbuf[pl.ds(0, L)] = col16 & _i32(GROUP - 1)
            m_sc = jnp.minimum(n_k - r * L, L)

            def ent_body(i, _3):
                clv = plsc.load_gather(clbuf, [_i32(i)])
                for kq in range(DD // L):
                    x = vbuf[i, pl.ds(kq * L, L)]
                    plsc.store_scatter(
                        cbuf, [iota + _i32(kq * L), clv], x)
                return 0

            lax.fori_loop(0, m_sc, ent_body, 0)
            return 0

        lax.fori_loop(0, nrounds, round_body, 0)

    my_nt = (NT_FULL - wid + NW - 1) // NW

    def start_in(k, cbuf, sem):
        t = wid + k * NW
        pltpu.async_copy(memT.at[:, pl.ds(t * GROUP, GROUP)], cbuf, sem)

    def wait_in(cbuf, sem):
        pltpu.make_async_copy(memT.at[:, pl.ds(0, GROUP)], cbuf, sem).wait()

    def start_out(k, cbuf, sem):
        t = wid + k * NW
        pltpu.async_copy(cbuf, outT.at[:, pl.ds(t * GROUP, GROUP)], sem)

    def wait_out(cbuf, sem):
        pltpu.make_async_copy(cbuf, outT.at[:, pl.ds(0, GROUP)], sem).wait()

    def when(cond, fn):
        def b(_, __):
            fn()
            return 0

        lax.fori_loop(0, jnp.where(cond, 1, 0), b, 0)

    start_in(0, chunk, sem_i)
    npairs = (my_nt + 1) // 2

    def pair_body(j, _):
        k0 = j * 2
        k1 = k0 + 1
        when((j > 0) & (k1 < my_nt), lambda: wait_out(chunk2, sem_o2))
        when(k1 < my_nt, lambda: start_in(k1, chunk2, sem_i2))
        wait_in(chunk, sem_i)
        apply_updates(k0, chunk)
        start_out(k0, chunk, sem_o)

        def do_b():
            wait_in(chunk2, sem_i2)
            apply_updates(k1, chunk2)
            start_out(k1, chunk2, sem_o2)

        when(k1 < my_nt, do_b)

        def prefetch_a():
            wait_out(chunk, sem_o)
            start_in(k0 + 2, chunk, sem_i)

        when(k0 + 2 < my_nt, prefetch_a)
        return 0

    lax.fori_loop(0, npairs, pair_body, 0)
    wait_out(chunk, sem_o)
    when(my_nt >= 2, lambda: wait_out(chunk2, sem_o2))

    def part_body(_, _2):
        pltpu.async_copy(memT.at[:, pl.ds(PART_BASE, PART_W)], chunk64,
                         sem_i).wait()
        apply_updates(PART_BIN, chunk64)
        pltpu.async_copy(chunk64, outT.at[:, pl.ds(PART_BASE, PART_W)],
                         sem_o).wait()
        return 0

    is_owner = jnp.where(wid == PART_OWNER, 1, 0)
    lax.fori_loop(0, is_owner, part_body, 0)


def _make_sc_update():
    mesh = plsc.VectorSubcoreMesh(core_axis_name="c", subcore_axis_name="s")
    return pl.kernel(
        _sc_body,
        out_type=jax.ShapeDtypeStruct((DD, MC), jnp.float32),
        mesh=mesh,
        compiler_params=pltpu.CompilerParams(needs_layout_passes=False),
        scratch_types=[
            pltpu.VMEM((SEC,), jnp.int32),
            pltpu.VMEM((CAP,), jnp.int32),
            pltpu.VMEM((CAP,), jnp.int32),
            pltpu.VMEM((CAP,), jnp.int32),
            pltpu.VMEM((CAP,), jnp.int32),
            pltpu.VMEM((NBINS,), jnp.int32),
            pltpu.VMEM((NBINS,), jnp.int32),
            pltpu.VMEM((NBINS,), jnp.int32),
            pltpu.VMEM((DD, GROUP), jnp.float32),
            pltpu.VMEM((DD, GROUP), jnp.float32),
            pltpu.VMEM((DD, PART_W), jnp.float32),
            pltpu.VMEM((L, 128), jnp.float32),
            pltpu.VMEM((L,), jnp.int32),
            pltpu.VMEM((L,), jnp.int32),
            pltpu.SemaphoreType.DMA,
            pltpu.SemaphoreType.DMA,
            pltpu.SemaphoreType.DMA,
            pltpu.SemaphoreType.DMA,
            pltpu.SemaphoreType.DMA,
        ],
    )


@jax.jit
def kernel(mem, idx, val):
    idx32 = idx.astype(jnp.int32)
    memT = mem.T
    val128 = jnp.pad(val, ((0, 0), (0, 128 - DD)))
    outT = _make_sc_update()(memT, idx32, val128)
    return outT.T


# speedup vs baseline: 15.8384x; 1.0854x over previous
import jax
import jax.numpy as jnp
from jax import lax
from jax.experimental import pallas as pl
from jax.experimental.pallas import tpu as pltpu
from jax.experimental.pallas import tpu_sc as plsc

NC = 2
NS = 16
NW = NC * NS
L = 16

MC = 1000000
DD = 64
BB = 65536
GROUP = 512
NT_FULL = MC // GROUP
PART_BASE = NT_FULL * GROUP
PART_W = MC - PART_BASE
PART_OWNER = NT_FULL % NW
PART_BIN = NT_FULL // NW
OW_SHIFT = 9
BIN_SHIFT = 14
SEC = 8192
NSEC = BB // SEC
CAP = 6144
PRE = 64
NBINS = 64


def _i32(x):
    return jnp.full((L,), x, jnp.int32)


def _sc_body(memT, idx_hbm, val_hbm, outT,
             idx_buf, tgt_c, pos_c, binned_t, binned_p,
             runhist, binbase, nextfree, chunk, chunk2, chunk64, vbuf,
             st16, clbuf, st_a, st_b, gbuf_a, gbuf_b,
             sem_i, sem_o, sem_i2, sem_o2, sem_g, sem_ga, sem_gb):
    wid = lax.axis_index("s") * NC + lax.axis_index("c")
    iota = lax.iota(jnp.int32, L)
    widv = _i32(wid)

    def sec_body(s, cv):
        pltpu.sync_copy(idx_hbm.at[pl.ds(s * SEC, SEC)], idx_buf)

        def scan_body(i, cv):
            for q in range(4):
                off = i * 4 * L + q * L
                v = idx_buf[pl.ds(off, L)]
                bvec = _i32(s * SEC) + _i32(off) + iota
                m = (lax.shift_right_logical(v, _i32(OW_SHIFT))
                     & _i32(NW - 1)) == widv
                m = m & (cv < _i32(CAP - L))
                mi = jnp.where(m, _i32(1), _i32(0))
                dest = cv + plsc.cumsum(mi) - mi
                plsc.store_scatter(tgt_c, [dest], v, mask=m)
                plsc.store_scatter(pos_c, [dest], bvec, mask=m)
                cv = cv + plsc.all_reduce_population_count(m)
            return cv

        return lax.fori_loop(0, SEC // (4 * L), scan_body, cv)

    cnt_vec = lax.fori_loop(0, NSEC, sec_body, _i32(0))
    cnt = jnp.max(cnt_vec)
    nv = (cnt + L - 1) // L

    def zb(j, _):
        runhist[pl.ds(j * L, L)] = _i32(0)
        return 0

    lax.fori_loop(0, NBINS // L, zb, 0)

    def hist_body(i, _):
        base = i * L
        t = tgt_c[pl.ds(base, L)]
        valid = (_i32(base) + iota) < _i32(cnt)
        binv = jnp.clip(lax.shift_right_logical(t, _i32(BIN_SHIFT)), 0,
                        NBINS - 1)
        rc, lastm = plsc.scan_count(binv, valid)
        plsc.addupdate_scatter(runhist, [binv], rc, mask=lastm & valid)
        return 0

    lax.fori_loop(0, nv, hist_body, 0)

    def scan_bins(j, run):
        h = runhist[pl.ds(j * L, L)]
        c = plsc.cumsum(h)
        binbase[pl.ds(j * L, L)] = _i32(run) + c - h
        return run + jnp.sum(h)

    lax.fori_loop(0, NBINS // L, scan_bins, jnp.int32(0))

    def cpnf(j, _):
        nextfree[pl.ds(j * L, L)] = _i32(0)
        return 0

    lax.fori_loop(0, NBINS // L, cpnf, 0)

    def place_body(i, _):
        base = i * L
        t = tgt_c[pl.ds(base, L)]
        p = pos_c[pl.ds(base, L)]
        valid = (_i32(base) + iota) < _i32(cnt)
        binv = jnp.clip(lax.shift_right_logical(t, _i32(BIN_SHIFT)), 0,
                        NBINS - 1)
        rc, lastm = plsc.scan_count(binv, valid)
        run = plsc.load_gather(nextfree, [binv], mask=valid)
        bb = plsc.load_gather(binbase, [binv], mask=valid)
        dest = bb + run + rc - _i32(1)
        plsc.store_scatter(binned_t, [dest], t, mask=valid)
        plsc.store_scatter(binned_p, [dest], p, mask=valid)
        plsc.addupdate_scatter(nextfree, [binv], rc, mask=lastm & valid)
        return 0

    lax.fori_loop(0, nv, place_body, 0)

    def bin_bounds(k):
        hb = (k // L) * L
        hv = runhist[pl.ds(hb, L)]
        bv = binbase[pl.ds(hb, L)]
        selk = iota == _i32(k - hb)
        n_k = jnp.sum(jnp.where(selk, hv, _i32(0)))
        b_k = jnp.sum(jnp.where(selk, bv, _i32(0)))
        return n_k, b_k

    def prefetch_gather(k, stbuf, gbuf, semx):
        n_k, b_k = bin_bounds(k)
        nm1 = jnp.maximum(n_k - 1, 0)
        for q in range(PRE // L):
            li = _i32(b_k) + jnp.minimum(_i32(q * L) + iota, _i32(nm1))
            pb = plsc.load_gather(binned_p, [li])
            stbuf[pl.ds(q * L, L)] = jnp.clip(pb, 0, BB - 1)
        pltpu.async_copy(val_hbm.at[stbuf], gbuf, semx)

    def apply_updates(k, cbuf, stbuf, gbuf, semx):
        n_k, b_k = bin_bounds(k)
        pltpu.make_async_copy(val_hbm.at[stbuf], gbuf, semx).wait()
        nmain = jnp.minimum(n_k, PRE)

        def ent_body(i, _3):
            clv = (plsc.load_gather(binned_t, [_i32(b_k) + _i32(i)])
                   & _i32(GROUP - 1))
            for kq in range(DD // L):
                x = gbuf[i, pl.ds(kq * L, L)]
                plsc.store_scatter(cbuf, [iota + _i32(kq * L), clv], x)
            return 0

        lax.fori_loop(0, nmain, ent_body, 0)

        nrounds = (n_k + L - 1) // L

        def round_body(r, _2):
            off = b_k + r * L
            li = _i32(off) + jnp.minimum(iota, _i32(n_k - r * L - 1))
            col16 = plsc.load_gather(binned_t, [li])
            pos16 = plsc.load_gather(binned_p, [li])
            st16[pl.ds(0, L)] = pos16
            pltpu.async_copy(val_hbm.at[st16], vbuf, sem_g).wait()
            clbuf[pl.ds(0, L)] = col16 & _i32(GROUP - 1)
            m_sc = jnp.minimum(n_k - r * L, L)

            def ent2(i, _3):
                clv = plsc.load_gather(clbuf, [_i32(i)])
                for kq in range(DD // L):
                    x = vbuf[i, pl.ds(kq * L, L)]
                    plsc.store_scatter(
                        cbuf, [iota + _i32(kq * L), clv], x)
                return 0

            lax.fori_loop(0, m_sc, ent2, 0)
            return 0

        lax.fori_loop(PRE // L, nrounds, round_body, 0)

    my_nt = (NT_FULL - wid + NW - 1) // NW

    def start_in(k, cbuf, sem):
        t = wid + k * NW
        pltpu.async_copy(memT.at[:, pl.ds(t * GROUP, GROUP)], cbuf, sem)

    def wait_in(cbuf, sem):
        pltpu.make_async_copy(memT.at[:, pl.ds(0, GROUP)], cbuf, sem).wait()

    def start_out(k, cbuf, sem):
        t = wid + k * NW
        pltpu.async_copy(cbuf, outT.at[:, pl.ds(t * GROUP, GROUP)], sem)

    def wait_out(cbuf, sem):
        pltpu.make_async_copy(cbuf, outT.at[:, pl.ds(0, GROUP)], sem).wait()

    def when(cond, fn):
        def b(_, __):
            fn()
            return 0

        lax.fori_loop(0, jnp.where(cond, 1, 0), b, 0)

    start_in(0, chunk, sem_i)
    npairs = (my_nt + 1) // 2

    def pair_body(j, _):
        k0 = j * 2
        k1 = k0 + 1
        when((j > 0) & (k1 < my_nt), lambda: wait_out(chunk2, sem_o2))
        when(k1 < my_nt, lambda: start_in(k1, chunk2, sem_i2))
        prefetch_gather(k0, st_a, gbuf_a, sem_ga)
        when(k1 < my_nt, lambda: prefetch_gather(k1, st_b, gbuf_b, sem_gb))
        wait_in(chunk, sem_i)
        apply_updates(k0, chunk, st_a, gbuf_a, sem_ga)
        start_out(k0, chunk, sem_o)

        def do_b():
            wait_in(chunk2, sem_i2)
            apply_updates(k1, chunk2, st_b, gbuf_b, sem_gb)
            start_out(k1, chunk2, sem_o2)

        when(k1 < my_nt, do_b)

        def prefetch_a():
            wait_out(chunk, sem_o)
            start_in(k0 + 2, chunk, sem_i)

        when(k0 + 2 < my_nt, prefetch_a)
        return 0

    lax.fori_loop(0, npairs, pair_body, 0)
    wait_out(chunk, sem_o)
    when(my_nt >= 2, lambda: wait_out(chunk2, sem_o2))

    def part_body(_, _2):
        prefetch_gather(PART_BIN, st_a, gbuf_a, sem_ga)
        pltpu.async_copy(memT.at[:, pl.ds(PART_BASE, PART_W)], chunk64,
                         sem_i).wait()
        apply_updates(PART_BIN, chunk64, st_a, gbuf_a, sem_ga)
        pltpu.async_copy(chunk64, outT.at[:, pl.ds(PART_BASE, PART_W)],
                         sem_o).wait()
        return 0

    is_owner = jnp.where(wid == PART_OWNER, 1, 0)
    lax.fori_loop(0, is_owner, part_body, 0)


def _make_sc_update():
    mesh = plsc.VectorSubcoreMesh(core_axis_name="c", subcore_axis_name="s")
    return pl.kernel(
        _sc_body,
        out_type=jax.ShapeDtypeStruct((DD, MC), jnp.float32),
        mesh=mesh,
        compiler_params=pltpu.CompilerParams(needs_layout_passes=False),
        scratch_types=[
            pltpu.VMEM((SEC,), jnp.int32),
            pltpu.VMEM((CAP,), jnp.int32),
            pltpu.VMEM((CAP,), jnp.int32),
            pltpu.VMEM((CAP,), jnp.int32),
            pltpu.VMEM((CAP,), jnp.int32),
            pltpu.VMEM((NBINS,), jnp.int32),
            pltpu.VMEM((NBINS,), jnp.int32),
            pltpu.VMEM((NBINS,), jnp.int32),
            pltpu.VMEM((DD, GROUP), jnp.float32),
            pltpu.VMEM((DD, GROUP), jnp.float32),
            pltpu.VMEM((DD, PART_W), jnp.float32),
            pltpu.VMEM((L, 128), jnp.float32),
            pltpu.VMEM((L,), jnp.int32),
            pltpu.VMEM((L,), jnp.int32),
            pltpu.VMEM((PRE,), jnp.int32),
            pltpu.VMEM((PRE,), jnp.int32),
            pltpu.VMEM((PRE, 128), jnp.float32),
            pltpu.VMEM((PRE, 128), jnp.float32),
            pltpu.SemaphoreType.DMA,
            pltpu.SemaphoreType.DMA,
            pltpu.SemaphoreType.DMA,
            pltpu.SemaphoreType.DMA,
            pltpu.SemaphoreType.DMA,
            pltpu.SemaphoreType.DMA,
            pltpu.SemaphoreType.DMA,
        ],
    )


@jax.jit
def kernel(mem, idx, val):
    idx32 = idx.astype(jnp.int32)
    memT = mem.T
    val128 = jnp.pad(val, ((0, 0), (0, 128 - DD)))
    outT = _make_sc_update()(memT, idx32, val128)
    return outT.T


# speedup vs baseline: 15.8681x; 1.0019x over previous
import jax
import jax.numpy as jnp
from jax import lax
from jax.experimental import pallas as pl
from jax.experimental.pallas import tpu as pltpu
from jax.experimental.pallas import tpu_sc as plsc

NC = 2
NS = 16
NW = NC * NS
L = 16

MC = 1000000
DD = 64
BB = 65536
GROUP = 512
NT_FULL = MC // GROUP
PART_BASE = NT_FULL * GROUP
PART_W = MC - PART_BASE
PART_OWNER = NT_FULL % NW
PART_BIN = NT_FULL // NW
OW_SHIFT = 9
BIN_SHIFT = 14
SEC = 8192
NSEC = BB // SEC
CAP = 6144
PRE = 64
NBINS = 64


def _i32(x):
    return jnp.full((L,), x, jnp.int32)


def _sc_body(memT, idx_hbm, val_hbm, outT,
             idx_buf, tgt_c, pos_c, binned_t, binned_p,
             runhist, binbase, nextfree, chunk, chunk2, chunk64, vbuf,
             st16, clbuf, st_a, st_b, gbuf_a, gbuf_b,
             sem_i, sem_o, sem_i2, sem_o2, sem_g, sem_ga, sem_gb):
    wid = lax.axis_index("s") * NC + lax.axis_index("c")
    iota = lax.iota(jnp.int32, L)
    widv = _i32(wid)

    def sec_body(s, cv):
        pltpu.sync_copy(idx_hbm.at[pl.ds(s * SEC, SEC)], idx_buf)

        def scan_body(i, cv):
            for q in range(4):
                off = i * 4 * L + q * L
                v = idx_buf[pl.ds(off, L)]
                bvec = _i32(s * SEC) + _i32(off) + iota
                m = (lax.shift_right_logical(v, _i32(OW_SHIFT))
                     & _i32(NW - 1)) == widv
                m = m & (cv < _i32(CAP - L))
                mi = jnp.where(m, _i32(1), _i32(0))
                dest = cv + plsc.cumsum(mi) - mi
                plsc.store_scatter(tgt_c, [dest], v, mask=m)
                plsc.store_scatter(pos_c, [dest], bvec, mask=m)
                cv = cv + plsc.all_reduce_population_count(m)
            return cv

        return lax.fori_loop(0, SEC // (4 * L), scan_body, cv)

    cnt_vec = lax.fori_loop(0, NSEC, sec_body, _i32(0))
    cnt = jnp.max(cnt_vec)
    nv = (cnt + L - 1) // L

    def zb(j, _):
        runhist[pl.ds(j * L, L)] = _i32(0)
        return 0

    lax.fori_loop(0, NBINS // L, zb, 0)

    def hist_body(i, _):
        base = i * L
        t = tgt_c[pl.ds(base, L)]
        valid = (_i32(base) + iota) < _i32(cnt)
        binv = jnp.clip(lax.shift_right_logical(t, _i32(BIN_SHIFT)), 0,
                        NBINS - 1)
        rc, lastm = plsc.scan_count(binv, valid)
        plsc.addupdate_scatter(runhist, [binv], rc, mask=lastm & valid)
        return 0

    lax.fori_loop(0, nv, hist_body, 0)

    def scan_bins(j, run):
        h = runhist[pl.ds(j * L, L)]
        c = plsc.cumsum(h)
        binbase[pl.ds(j * L, L)] = _i32(run) + c - h
        return run + jnp.sum(h)

    lax.fori_loop(0, NBINS // L, scan_bins, jnp.int32(0))

    def cpnf(j, _):
        nextfree[pl.ds(j * L, L)] = _i32(0)
        return 0

    lax.fori_loop(0, NBINS // L, cpnf, 0)

    def place_body(i, _):
        base = i * L
        t = tgt_c[pl.ds(base, L)]
        p = pos_c[pl.ds(base, L)]
        valid = (_i32(base) + iota) < _i32(cnt)
        binv = jnp.clip(lax.shift_right_logical(t, _i32(BIN_SHIFT)), 0,
                        NBINS - 1)
        rc, lastm = plsc.scan_count(binv, valid)
        run = plsc.load_gather(nextfree, [binv], mask=valid)
        bb = plsc.load_gather(binbase, [binv], mask=valid)
        dest = bb + run + rc - _i32(1)
        plsc.store_scatter(binned_t, [dest], t, mask=valid)
        plsc.store_scatter(binned_p, [dest], p, mask=valid)
        plsc.addupdate_scatter(nextfree, [binv], rc, mask=lastm & valid)
        return 0

    lax.fori_loop(0, nv, place_body, 0)

    def bin_bounds(k):
        hb = (k // L) * L
        hv = runhist[pl.ds(hb, L)]
        bv = binbase[pl.ds(hb, L)]
        selk = iota == _i32(k - hb)
        n_k = jnp.sum(jnp.where(selk, hv, _i32(0)))
        b_k = jnp.sum(jnp.where(selk, bv, _i32(0)))
        return n_k, b_k

    def prefetch_gather(k, stbuf, gbuf, semx):
        n_k, b_k = bin_bounds(k)
        nm1 = jnp.maximum(n_k - 1, 0)
        for q in range(PRE // L):
            li = _i32(b_k) + jnp.minimum(_i32(q * L) + iota, _i32(nm1))
            pb = plsc.load_gather(binned_p, [li])
            stbuf[pl.ds(q * L, L)] = jnp.clip(pb, 0, BB - 1)
        pltpu.async_copy(val_hbm.at[stbuf], gbuf, semx)

    def apply_updates(k, cbuf, stbuf, gbuf, semx):
        n_k, b_k = bin_bounds(k)
        pltpu.make_async_copy(val_hbm.at[stbuf], gbuf, semx).wait()
        nmain = jnp.minimum(n_k, PRE)

        def ent_body4(i4, _3):
            for u in range(4):
                i = i4 * 4 + u
                clv = (plsc.load_gather(binned_t, [_i32(b_k) + _i32(i)])
                       & _i32(GROUP - 1))
                for kq in range(DD // L):
                    x = gbuf[i, pl.ds(kq * L, L)]
                    plsc.store_scatter(cbuf, [iota + _i32(kq * L), clv], x)
            return 0

        lax.fori_loop(0, nmain // 4, ent_body4, 0)

        def ent_body(i, _3):
            clv = (plsc.load_gather(binned_t, [_i32(b_k) + _i32(i)])
                   & _i32(GROUP - 1))
            for kq in range(DD // L):
                x = gbuf[i, pl.ds(kq * L, L)]
                plsc.store_scatter(cbuf, [iota + _i32(kq * L), clv], x)
            return 0

        lax.fori_loop((nmain // 4) * 4, nmain, ent_body, 0)

        nrounds = (n_k + L - 1) // L

        def round_body(r, _2):
            off = b_k + r * L
            li = _i32(off) + jnp.minimum(iota, _i32(n_k - r * L - 1))
            col16 = plsc.load_gather(binned_t, [li])
            pos16 = plsc.load_gather(binned_p, [li])
            st16[pl.ds(0, L)] = pos16
            pltpu.async_copy(val_hbm.at[st16], vbuf, sem_g).wait()
            clbuf[pl.ds(0, L)] = col16 & _i32(GROUP - 1)
            m_sc = jnp.minimum(n_k - r * L, L)

            def ent2(i, _3):
                clv = plsc.load_gather(clbuf, [_i32(i)])
                for kq in range(DD // L):
                    x = vbuf[i, pl.ds(kq * L, L)]
                    plsc.store_scatter(
                        cbuf, [iota + _i32(kq * L), clv], x)
                return 0

            lax.fori_loop(0, m_sc, ent2, 0)
            return 0

        lax.fori_loop(PRE // L, nrounds, round_body, 0)

    my_nt = (NT_FULL - wid + NW - 1) // NW

    def start_in(k, cbuf, sem):
        t = wid + k * NW
        pltpu.async_copy(memT.at[:, pl.ds(t * GROUP, GROUP)], cbuf, sem)

    def wait_in(cbuf, sem):
        pltpu.make_async_copy(memT.at[:, pl.ds(0, GROUP)], cbuf, sem).wait()

    def start_out(k, cbuf, sem):
        t = wid + k * NW
        pltpu.async_copy(cbuf, outT.at[:, pl.ds(t * GROUP, GROUP)], sem)

    def wait_out(cbuf, sem):
        pltpu.make_async_copy(cbuf, outT.at[:, pl.ds(0, GROUP)], sem).wait()

    def when(cond, fn):
        def b(_, __):
            fn()
            return 0

        lax.fori_loop(0, jnp.where(cond, 1, 0), b, 0)

    start_in(0, chunk, sem_i)
    npairs = (my_nt + 1) // 2

    def pair_body(j, _):
        k0 = j * 2
        k1 = k0 + 1
        when((j > 0) & (k1 < my_nt), lambda: wait_out(chunk2, sem_o2))
        when(k1 < my_nt, lambda: start_in(k1, chunk2, sem_i2))
        prefetch_gather(k0, st_a, gbuf_a, sem_ga)
        when(k1 < my_nt, lambda: prefetch_gather(k1, st_b, gbuf_b, sem_gb))
        wait_in(chunk, sem_i)
        apply_updates(k0, chunk, st_a, gbuf_a, sem_ga)
        start_out(k0, chunk, sem_o)

        def do_b():
            wait_in(chunk2, sem_i2)
            apply_updates(k1, chunk2, st_b, gbuf_b, sem_gb)
            start_out(k1, chunk2, sem_o2)

        when(k1 < my_nt, do_b)

        def prefetch_a():
            wait_out(chunk, sem_o)
            start_in(k0 + 2, chunk, sem_i)

        when(k0 + 2 < my_nt, prefetch_a)
        return 0

    lax.fori_loop(0, npairs, pair_body, 0)
    wait_out(chunk, sem_o)
    when(my_nt >= 2, lambda: wait_out(chunk2, sem_o2))

    def part_body(_, _2):
        prefetch_gather(PART_BIN, st_a, gbuf_a, sem_ga)
        pltpu.async_copy(memT.at[:, pl.ds(PART_BASE, PART_W)], chunk64,
                         sem_i).wait()
        apply_updates(PART_BIN, chunk64, st_a, gbuf_a, sem_ga)
        pltpu.async_copy(chunk64, outT.at[:, pl.ds(PART_BASE, PART_W)],
                         sem_o).wait()
        return 0

    is_owner = jnp.where(wid == PART_OWNER, 1, 0)
    lax.fori_loop(0, is_owner, part_body, 0)


def _make_sc_update():
    mesh = plsc.VectorSubcoreMesh(core_axis_name="c", subcore_axis_name="s")
    return pl.kernel(
        _sc_body,
        out_type=jax.ShapeDtypeStruct((DD, MC), jnp.float32),
        mesh=mesh,
        compiler_params=pltpu.CompilerParams(needs_layout_passes=False),
        scratch_types=[
            pltpu.VMEM((SEC,), jnp.int32),
            pltpu.VMEM((CAP,), jnp.int32),
            pltpu.VMEM((CAP,), jnp.int32),
            pltpu.VMEM((CAP,), jnp.int32),
            pltpu.VMEM((CAP,), jnp.int32),
            pltpu.VMEM((NBINS,), jnp.int32),
            pltpu.VMEM((NBINS,), jnp.int32),
            pltpu.VMEM((NBINS,), jnp.int32),
            pltpu.VMEM((DD, GROUP), jnp.float32),
            pltpu.VMEM((DD, GROUP), jnp.float32),
            pltpu.VMEM((DD, PART_W), jnp.float32),
            pltpu.VMEM((L, 128), jnp.float32),
            pltpu.VMEM((L,), jnp.int32),
            pltpu.VMEM((L,), jnp.int32),
            pltpu.VMEM((PRE,), jnp.int32),
            pltpu.VMEM((PRE,), jnp.int32),
            pltpu.VMEM((PRE, 128), jnp.float32),
            pltpu.VMEM((PRE, 128), jnp.float32),
            pltpu.SemaphoreType.DMA,
            pltpu.SemaphoreType.DMA,
            pltpu.SemaphoreType.DMA,
            pltpu.SemaphoreType.DMA,
            pltpu.SemaphoreType.DMA,
            pltpu.SemaphoreType.DMA,
            pltpu.SemaphoreType.DMA,
        ],
    )


@jax.jit
def kernel(mem, idx, val):
    idx32 = idx.astype(jnp.int32)
    memT = mem.T
    val128 = jnp.pad(val, ((0, 0), (0, 128 - DD)))
    outT = _make_sc_update()(memT, idx32, val128)
    return outT.T
